# baseline (speedup 1.0000x reference)
import functools

import numpy as np
import jax
import jax.numpy as jnp
from jax import lax
from jax.experimental import pallas as pl
from jax.experimental.pallas import tpu as pltpu
from jax.experimental.pallas import tpu_sc as plsc

_TWO_PI = np.float32(2.0 * np.pi)
_INV_SQRT2 = np.float32(0.7071067811865476)
_EPS = np.float32(1e-6)


def _gelu(x):
    return 0.5 * x * (1.0 + lax.erf(x * _INV_SQRT2))


def _pe_body(c_ref, pe_ref, o_ref):
    t = c_ref[...] + 0.5
    t = 2.0 * (t * np.float32(1.0 / 1024.0)) - 1.0
    t = t.astype(jnp.bfloat16).astype(jnp.float32)
    peb = pe_ref[...].astype(jnp.bfloat16).astype(jnp.float32)
    ang = t[:, 0:1] * peb[0:1, :] + t[:, 1:2] * peb[1:2, :]
    ang = ang * _TWO_PI
    o_ref[:, 0:128] = jnp.sin(ang)
    o_ref[:, 128:256] = jnp.cos(ang)


def _pe_encode_pallas(coords_flat, pe_matrix):
    n = coords_flat.shape[0]
    return pl.pallas_call(
        _pe_body,
        out_shape=jax.ShapeDtypeStruct((n, 256), jnp.float32),
    )(coords_flat, pe_matrix)


def _sc_sparse(raw, labels, table, badd):
    B = raw.shape[0]
    mesh = plsc.VectorSubcoreMesh(core_axis_name="c", subcore_axis_name="s")

    @functools.partial(
        pl.kernel,
        mesh=mesh,
        out_type=jax.ShapeDtypeStruct((B, 4608), jnp.float32),
        scratch_types=[
            pltpu.VMEM((16,), jnp.int32),
            pltpu.VMEM((4608,), jnp.float32),
            pltpu.VMEM((768,), jnp.float32),
            pltpu.VMEM((512,), jnp.float32),
        ],
    )
    def body(raw_hbm, labels_hbm, table_hbm, badd_hbm, out_hbm,
             labs_v, buf_v, tab_v, badd_v):
        wid = lax.axis_index("s") * 2 + lax.axis_index("c")
        pltpu.sync_copy(labels_hbm.at[wid], labs_v)
        pltpu.sync_copy(raw_hbm.at[wid], buf_v)
        pltpu.sync_copy(table_hbm, tab_v)
        pltpu.sync_copy(badd_hbm, badd_v)
        labs = labs_v[...]
        one = np.float32(1.0)
        zero = np.float32(0.0)
        keep = [jnp.where(labs[p] == -1, zero, one) for p in range(16)]
        mn = [jnp.where(labs[p] == -1, one, zero) for p in range(16)]
        m0 = [jnp.where(labs[p] == 0, one, zero) for p in range(16)]
        m1 = [jnp.where(labs[p] == 1, one, zero) for p in range(16)]
        for c in range(16):
            t_nap = tab_v[pl.ds(c * 16, 16)]
            t_pe0 = tab_v[pl.ds(256 + c * 16, 16)]
            t_pe1 = tab_v[pl.ds(512 + c * 16, 16)]
            for p in range(16):
                sl = pl.ds(p * 256 + c * 16, 16)
                buf_v[sl] = (buf_v[sl] * keep[p] + t_nap * mn[p]
                             + t_pe0 * m0[p] + t_pe1 * m1[p])
        for j in range(2):
            for c in range(16):
                sl = pl.ds((16 + j) * 256 + c * 16, 16)
                buf_v[sl] = buf_v[sl] + badd_v[pl.ds(j * 256 + c * 16, 16)]
        pltpu.sync_copy(buf_v, out_hbm.at[wid])

    return body(raw, labels, table, badd)


def _mmb(a, b):
    return jnp.dot(a, b.astype(jnp.bfloat16),
                   preferred_element_type=jnp.float32)


def _dense_body(nsb, b_off, *refs):
    (x_ref, m1_ref, b1_ref, g1_ref, w1_ref, v1_ref,
     m2_ref, b2_ref, w2_ref, v2_ref, w3_ref, b3_ref) = refs[:12]
    o_hbm, obuf, sem = refs[-3:]
    b = pl.program_id(0)
    s = pl.program_id(1)
    nbuf = obuf.shape[0]
    sblk = obuf.shape[2]
    g = b * nsb + s
    slot = lax.rem(g, nbuf)
    X = x_ref[0]
    h = _mmb(m1_ref[...], X) + b1_ref[...]
    m = _mmb(g1_ref[...], h)
    d = h - m
    v = _mmb(g1_ref[...], d * d)
    h = d * lax.rsqrt(v + _EPS) * w1_ref[...] + v1_ref[...]
    h = _gelu(h)
    h = _mmb(m2_ref[...], h) + b2_ref[...]
    mu = jnp.mean(h, axis=0, keepdims=True)
    d2 = h - mu
    var = jnp.mean(d2 * d2, axis=0, keepdims=True)
    h = d2 * lax.rsqrt(var + _EPS) * w2_ref[...] + v2_ref[...]
    h = _gelu(h)

    dst = o_hbm.at[b + b_off, :, pl.ds(s * sblk, sblk)]

    @pl.when(g >= nbuf)
    def _():
        pltpu.make_async_copy(obuf.at[slot], dst, sem.at[slot]).wait()

    obuf[slot] = _mmb(w3_ref[...], h) + b3_ref[...]
    pltpu.make_async_copy(obuf.at[slot], dst, sem.at[slot]).start()

    total = pl.num_programs(0) * nsb

    @pl.when(g == total - 1)
    def _():
        for k in range(min(nbuf, total)):
            pltpu.make_async_copy(obuf.at[k], dst, sem.at[k]).wait()


def _dense_pallas(x, m1, b1, g1, w1, v1, m2, b2, w2, v2, w3, b3,
                  total_b, b_off=0, prev=None, sblk=4096, nbuf=2):
    Bh = x.shape[0]
    S = x.shape[2]
    nsb = S // sblk
    const = lambda shape: pl.BlockSpec(shape, lambda b, s: (0, 0))
    in_specs = [
        pl.BlockSpec((1, 16, sblk), lambda b, s: (b, 0, s)),
        const((16, 16)), const((16, 1)), const((16, 16)),
        const((16, 1)), const((16, 1)),
        const((16, 16)), const((16, 1)), const((16, 1)), const((16, 1)),
        const((256, 16)), const((256, 1)),
    ]
    args = [x, m1, b1, g1, w1, v1, m2, b2, w2, v2, w3, b3]
    kwargs = {}
    if prev is not None:
        in_specs.append(pl.BlockSpec(memory_space=pl.ANY))
        args.append(prev)
        kwargs["input_output_aliases"] = {12: 0}
    return pl.pallas_call(
        functools.partial(_dense_body, nsb, b_off),
        grid=(Bh, nsb),
        in_specs=in_specs,
        out_specs=pl.BlockSpec(memory_space=pl.ANY),
        out_shape=jax.ShapeDtypeStruct((total_b, 256, S), jnp.float32),
        scratch_shapes=[
            pltpu.VMEM((nbuf, 256, sblk), jnp.float32),
            pltpu.SemaphoreType.DMA((nbuf,)),
        ],
        **kwargs,
    )(*args)


_S1 = np.zeros((4, 2, 2, 16), dtype=np.float32)
for _di in range(2):
    for _dj in range(2):
        for _u in range(2):
            for _v in range(2):
                _S1[_di * 2 + _dj, _u, _v, (2 * _di + _u) * 4 + (2 * _dj + _v)] = 1.0
_G1 = np.kron(np.eye(4, dtype=np.float32),
              np.full((4, 4), 0.25, dtype=np.float32))


def kernel(points, labels, boxes, masks, pe_matrix, point_embed0,
           point_embed1, point_embed2, point_embed3, not_a_point,
           conv1_w, conv1_b, ln1_w, ln1_b, conv2_w, conv2_b, ln2_w, ln2_b,
           conv3_w, conv3_b):
    B = points.shape[0]

    Bh = B // 2
    m4 = masks.reshape(B, 64, 4, 64, 4)
    xa = m4[:Bh].transpose(0, 2, 4, 1, 3).reshape(Bh, 16, 4096)
    xb = m4[Bh:].transpose(0, 2, 4, 1, 3).reshape(Bh, 16, 4096)

    m1 = jnp.einsum("cuv,xuvr->xcr", conv1_w[:, 0], jnp.asarray(_S1))
    m1 = m1.reshape(16, 16).astype(jnp.bfloat16)
    b1 = jnp.tile(conv1_b, 4)[:, None]
    w1 = jnp.tile(ln1_w, 4)[:, None]
    v1 = jnp.tile(ln1_b, 4)[:, None]
    m2 = conv2_w.transpose(0, 2, 3, 1).reshape(16, 16).astype(jnp.bfloat16)
    b2 = conv2_b[:, None]
    w2 = ln2_w[:, None]
    v2 = ln2_b[:, None]
    w3 = conv3_w[:, :, 0, 0].astype(jnp.bfloat16)
    b3 = conv3_b[:, None]
    g1 = jnp.asarray(_G1, dtype=jnp.bfloat16)
    d1 = _dense_pallas(xa, m1, b1, g1, w1, v1, m2, b2, w2, v2, w3, b3,
                       total_b=B, b_off=0)
    dense = _dense_pallas(xb, m1, b1, g1, w1, v1, m2, b2, w2, v2, w3, b3,
                          total_b=B, b_off=Bh, prev=d1)

    all_c = jnp.concatenate([points, boxes.reshape(B, 2, 2)], axis=1)
    raw = _pe_encode_pallas(all_c.reshape(B * 18, 2), pe_matrix)
    table = jnp.concatenate([not_a_point, point_embed0, point_embed1])
    badd = jnp.concatenate([point_embed2, point_embed3])
    sparse = _sc_sparse(raw.reshape(B, 18 * 256),
                        labels.astype(jnp.int32), table, badd)
    sparse = sparse.reshape(B, 18, 256)
    return sparse, dense.reshape(B, 256, 64, 64)


# speedup vs baseline: 1.3771x; 1.3771x over previous
import functools

import numpy as np
import jax
import jax.numpy as jnp
from jax import lax
from jax.experimental import pallas as pl
from jax.experimental.pallas import tpu as pltpu
from jax.experimental.pallas import tpu_sc as plsc

_TWO_PI = np.float32(2.0 * np.pi)
_INV_SQRT2 = np.float32(0.7071067811865476)
_EPS = np.float32(1e-6)


def _gelu(x):
    return 0.5 * x * (1.0 + lax.erf(x * _INV_SQRT2))


def _pe_body(c_ref, pe_ref, o_ref):
    t = c_ref[...] + 0.5
    t = 2.0 * (t * np.float32(1.0 / 1024.0)) - 1.0
    t = t.astype(jnp.bfloat16).astype(jnp.float32)
    peb = pe_ref[...].astype(jnp.bfloat16).astype(jnp.float32)
    ang = t[:, 0:1] * peb[0:1, :] + t[:, 1:2] * peb[1:2, :]
    ang = ang * _TWO_PI
    o_ref[:, 0:128] = jnp.sin(ang)
    o_ref[:, 128:256] = jnp.cos(ang)


def _pe_encode_pallas(coords_flat, pe_matrix):
    n = coords_flat.shape[0]
    return pl.pallas_call(
        _pe_body,
        out_shape=jax.ShapeDtypeStruct((n, 256), jnp.float32),
    )(coords_flat, pe_matrix)


def _sc_sparse(raw, labels, table, badd):
    B = raw.shape[0]
    mesh = plsc.VectorSubcoreMesh(core_axis_name="c", subcore_axis_name="s")

    @functools.partial(
        pl.kernel,
        mesh=mesh,
        out_type=jax.ShapeDtypeStruct((B, 4608), jnp.float32),
        scratch_types=[
            pltpu.VMEM((16,), jnp.int32),
            pltpu.VMEM((4608,), jnp.float32),
            pltpu.VMEM((768,), jnp.float32),
            pltpu.VMEM((512,), jnp.float32),
        ],
    )
    def body(raw_hbm, labels_hbm, table_hbm, badd_hbm, out_hbm,
             labs_v, buf_v, tab_v, badd_v):
        wid = lax.axis_index("s") * 2 + lax.axis_index("c")
        pltpu.sync_copy(labels_hbm.at[wid], labs_v)
        pltpu.sync_copy(raw_hbm.at[wid], buf_v)
        pltpu.sync_copy(table_hbm, tab_v)
        pltpu.sync_copy(badd_hbm, badd_v)
        labs = labs_v[...]
        one = np.float32(1.0)
        zero = np.float32(0.0)
        keep = [jnp.where(labs[p] == -1, zero, one) for p in range(16)]
        mn = [jnp.where(labs[p] == -1, one, zero) for p in range(16)]
        m0 = [jnp.where(labs[p] == 0, one, zero) for p in range(16)]
        m1 = [jnp.where(labs[p] == 1, one, zero) for p in range(16)]
        for c in range(16):
            t_nap = tab_v[pl.ds(c * 16, 16)]
            t_pe0 = tab_v[pl.ds(256 + c * 16, 16)]
            t_pe1 = tab_v[pl.ds(512 + c * 16, 16)]
            for p in range(16):
                sl = pl.ds(p * 256 + c * 16, 16)
                buf_v[sl] = (buf_v[sl] * keep[p] + t_nap * mn[p]
                             + t_pe0 * m0[p] + t_pe1 * m1[p])
        for j in range(2):
            for c in range(16):
                sl = pl.ds((16 + j) * 256 + c * 16, 16)
                buf_v[sl] = buf_v[sl] + badd_v[pl.ds(j * 256 + c * 16, 16)]
        pltpu.sync_copy(buf_v, out_hbm.at[wid])

    return body(raw, labels, table, badd)


def _mmb(a, b):
    return jnp.dot(a, b.astype(jnp.bfloat16),
                   preferred_element_type=jnp.float32)


def _dense_body(nsb, x_ref, m1_ref, b1_ref, g1_ref, w1_ref, v1_ref,
                m2_ref, b2_ref, w2_ref, v2_ref, w3_ref, b3_ref, o_hbm,
                obuf, sem):
    b = pl.program_id(0)
    s = pl.program_id(1)
    nbuf = obuf.shape[0]
    sblk = obuf.shape[2]
    g = b * nsb + s
    slot = lax.rem(g, nbuf)
    X = x_ref[0]
    h = _mmb(m1_ref[...], X) + b1_ref[...]
    m = _mmb(g1_ref[...], h)
    d = h - m
    v = _mmb(g1_ref[...], d * d)
    h = d * lax.rsqrt(v + _EPS) * w1_ref[...] + v1_ref[...]
    h = _gelu(h)
    h = _mmb(m2_ref[...], h) + b2_ref[...]
    mu = jnp.mean(h, axis=0, keepdims=True)
    d2 = h - mu
    var = jnp.mean(d2 * d2, axis=0, keepdims=True)
    h = d2 * lax.rsqrt(var + _EPS) * w2_ref[...] + v2_ref[...]
    h = _gelu(h)

    dst = o_hbm.at[b, :, pl.ds(s * sblk, sblk)]

    @pl.when(g >= nbuf)
    def _():
        pltpu.make_async_copy(obuf.at[slot], dst, sem.at[slot]).wait()

    obuf[slot] = _mmb(w3_ref[...], h) + b3_ref[...]
    pltpu.make_async_copy(obuf.at[slot], dst, sem.at[slot]).start()

    total = pl.num_programs(0) * nsb

    @pl.when(g == total - 1)
    def _():
        for k in range(min(nbuf, total)):
            pltpu.make_async_copy(obuf.at[k], dst, sem.at[k]).wait()


def _dense_pallas(x, m1, b1, g1, w1, v1, m2, b2, w2, v2, w3, b3,
                  sblk=4096, nbuf=2):
    B = x.shape[0]
    S = x.shape[2]
    nsb = S // sblk
    const = lambda shape: pl.BlockSpec(shape, lambda b, s: (0, 0))
    return pl.pallas_call(
        functools.partial(_dense_body, nsb),
        grid=(B, nsb),
        in_specs=[
            pl.BlockSpec((1, 16, sblk), lambda b, s: (b, 0, s)),
            const((16, 16)), const((16, 1)), const((16, 16)),
            const((16, 1)), const((16, 1)),
            const((16, 16)), const((16, 1)), const((16, 1)), const((16, 1)),
            const((256, 16)), const((256, 1)),
        ],
        out_specs=pl.BlockSpec(memory_space=pl.ANY),
        out_shape=jax.ShapeDtypeStruct((B, 256, S), jnp.float32),
        scratch_shapes=[
            pltpu.VMEM((nbuf, 256, sblk), jnp.float32),
            pltpu.SemaphoreType.DMA((nbuf,)),
        ],
    )(x, m1, b1, g1, w1, v1, m2, b2, w2, v2, w3, b3)


_S1 = np.zeros((4, 2, 2, 16), dtype=np.float32)
for _di in range(2):
    for _dj in range(2):
        for _u in range(2):
            for _v in range(2):
                _S1[_di * 2 + _dj, _u, _v, (2 * _di + _u) * 4 + (2 * _dj + _v)] = 1.0
_G1 = np.kron(np.eye(4, dtype=np.float32),
              np.full((4, 4), 0.25, dtype=np.float32))


def kernel(points, labels, boxes, masks, pe_matrix, point_embed0,
           point_embed1, point_embed2, point_embed3, not_a_point,
           conv1_w, conv1_b, ln1_w, ln1_b, conv2_w, conv2_b, ln2_w, ln2_b,
           conv3_w, conv3_b):
    B = points.shape[0]

    x = masks.reshape(B, 64, 4, 64, 4).transpose(0, 2, 4, 1, 3)
    x = x.reshape(B, 16, 4096)

    m1 = jnp.einsum("cuv,xuvr->xcr", conv1_w[:, 0], jnp.asarray(_S1))
    m1 = m1.reshape(16, 16).astype(jnp.bfloat16)
    b1 = jnp.tile(conv1_b, 4)[:, None]
    w1 = jnp.tile(ln1_w, 4)[:, None]
    v1 = jnp.tile(ln1_b, 4)[:, None]
    m2 = conv2_w.transpose(0, 2, 3, 1).reshape(16, 16).astype(jnp.bfloat16)
    b2 = conv2_b[:, None]
    w2 = ln2_w[:, None]
    v2 = ln2_b[:, None]
    w3 = conv3_w[:, :, 0, 0].astype(jnp.bfloat16)
    b3 = conv3_b[:, None]
    g1 = jnp.asarray(_G1, dtype=jnp.bfloat16)
    dense = _dense_pallas(x, m1, b1, g1, w1, v1, m2, b2, w2, v2, w3, b3)

    all_c = jnp.concatenate([points, boxes.reshape(B, 2, 2)], axis=1)
    raw = _pe_encode_pallas(all_c.reshape(B * 18, 2), pe_matrix)
    table = jnp.concatenate([not_a_point, point_embed0, point_embed1])
    badd = jnp.concatenate([point_embed2, point_embed3])
    sparse = _sc_sparse(raw.reshape(B, 18 * 256),
                        labels.astype(jnp.int32), table, badd)
    sparse = sparse.reshape(B, 18, 256)
    return sparse, dense.reshape(B, 256, 64, 64)


# speedup vs baseline: 1.3815x; 1.0032x over previous
import functools

import numpy as np
import jax
import jax.numpy as jnp
from jax import lax
from jax.experimental import pallas as pl
from jax.experimental.pallas import tpu as pltpu
from jax.experimental.pallas import tpu_sc as plsc

_TWO_PI = np.float32(2.0 * np.pi)
_INV_SQRT2 = np.float32(0.7071067811865476)
_EPS = np.float32(1e-6)


def _gelu(x):
    return 0.5 * x * (1.0 + lax.erf(x * _INV_SQRT2))


def _pe_body(c_ref, pe_ref, o_ref):
    t = c_ref[...] + 0.5
    t = 2.0 * (t * np.float32(1.0 / 1024.0)) - 1.0
    t = t.astype(jnp.bfloat16).astype(jnp.float32)
    peb = pe_ref[...].astype(jnp.bfloat16).astype(jnp.float32)
    ang = t[:, 0:1] * peb[0:1, :] + t[:, 1:2] * peb[1:2, :]
    ang = ang * _TWO_PI
    o_ref[:, 0:128] = jnp.sin(ang)
    o_ref[:, 128:256] = jnp.cos(ang)


def _pe_encode_pallas(coords_flat, pe_matrix):
    n = coords_flat.shape[0]
    return pl.pallas_call(
        _pe_body,
        out_shape=jax.ShapeDtypeStruct((n, 256), jnp.float32),
    )(coords_flat, pe_matrix)


def _sc_sparse(raw, labels, table, badd):
    B = raw.shape[0]
    mesh = plsc.VectorSubcoreMesh(core_axis_name="c", subcore_axis_name="s")

    @functools.partial(
        pl.kernel,
        mesh=mesh,
        out_type=jax.ShapeDtypeStruct((B, 4608), jnp.float32),
        scratch_types=[
            pltpu.VMEM((16,), jnp.int32),
            pltpu.VMEM((4608,), jnp.float32),
            pltpu.VMEM((768,), jnp.float32),
            pltpu.VMEM((512,), jnp.float32),
        ],
    )
    def body(raw_hbm, labels_hbm, table_hbm, badd_hbm, out_hbm,
             labs_v, buf_v, tab_v, badd_v):
        wid = lax.axis_index("s") * 2 + lax.axis_index("c")
        pltpu.sync_copy(labels_hbm.at[wid], labs_v)
        pltpu.sync_copy(raw_hbm.at[wid], buf_v)
        pltpu.sync_copy(table_hbm, tab_v)
        pltpu.sync_copy(badd_hbm, badd_v)
        labs = labs_v[...]
        one = np.float32(1.0)
        zero = np.float32(0.0)
        keep = [jnp.where(labs[p] == -1, zero, one) for p in range(16)]
        mn = [jnp.where(labs[p] == -1, one, zero) for p in range(16)]
        m0 = [jnp.where(labs[p] == 0, one, zero) for p in range(16)]
        m1 = [jnp.where(labs[p] == 1, one, zero) for p in range(16)]
        for c in range(16):
            t_nap = tab_v[pl.ds(c * 16, 16)]
            t_pe0 = tab_v[pl.ds(256 + c * 16, 16)]
            t_pe1 = tab_v[pl.ds(512 + c * 16, 16)]
            for p in range(16):
                sl = pl.ds(p * 256 + c * 16, 16)
                buf_v[sl] = (buf_v[sl] * keep[p] + t_nap * mn[p]
                             + t_pe0 * m0[p] + t_pe1 * m1[p])
        for j in range(2):
            for c in range(16):
                sl = pl.ds((16 + j) * 256 + c * 16, 16)
                buf_v[sl] = buf_v[sl] + badd_v[pl.ds(j * 256 + c * 16, 16)]
        pltpu.sync_copy(buf_v, out_hbm.at[wid])

    return body(raw, labels, table, badd)


def _mmb(a, b):
    return jnp.dot(a, b.astype(jnp.bfloat16),
                   preferred_element_type=jnp.float32)


def _dense_body(x_ref, m1_ref, b1_ref, g1_ref, w1_ref, v1_ref,
                m2_ref, b2_ref, w2_ref, v2_ref, w3_ref, b3_ref, o_hbm,
                obuf, sem):
    g = pl.program_id(0)
    nbuf = obuf.shape[0]
    bpg = obuf.shape[1]
    slot = lax.rem(g, nbuf)

    def one_batch(X):
        h = _mmb(m1_ref[...], X) + b1_ref[...]
        m = _mmb(g1_ref[...], h)
        d = h - m
        v = _mmb(g1_ref[...], d * d)
        h = d * lax.rsqrt(v + _EPS) * w1_ref[...] + v1_ref[...]
        h = _gelu(h)
        h = _mmb(m2_ref[...], h) + b2_ref[...]
        mu = jnp.mean(h, axis=0, keepdims=True)
        d2 = h - mu
        var = jnp.mean(d2 * d2, axis=0, keepdims=True)
        h = d2 * lax.rsqrt(var + _EPS) * w2_ref[...] + v2_ref[...]
        h = _gelu(h)
        return _mmb(w3_ref[...], h) + b3_ref[...]

    dst = o_hbm.at[pl.ds(g * bpg, bpg)]

    @pl.when(g >= nbuf)
    def _():
        pltpu.make_async_copy(obuf.at[slot], dst, sem.at[slot]).wait()

    for bi in range(bpg):
        obuf[slot, bi] = one_batch(x_ref[bi])
    pltpu.make_async_copy(obuf.at[slot], dst, sem.at[slot]).start()

    total = pl.num_programs(0)

    @pl.when(g == total - 1)
    def _():
        for k in range(min(nbuf, total)):
            pltpu.make_async_copy(obuf.at[k], dst, sem.at[k]).wait()


def _dense_pallas(x, m1, b1, g1, w1, v1, m2, b2, w2, v2, w3, b3,
                  bpg=2, nbuf=2):
    B = x.shape[0]
    S = x.shape[2]
    const = lambda shape: pl.BlockSpec(shape, lambda g: (0, 0))
    return pl.pallas_call(
        _dense_body,
        grid=(B // bpg,),
        in_specs=[
            pl.BlockSpec((bpg, 16, S), lambda g: (g, 0, 0)),
            const((16, 16)), const((16, 1)), const((16, 16)),
            const((16, 1)), const((16, 1)),
            const((16, 16)), const((16, 1)), const((16, 1)), const((16, 1)),
            const((256, 16)), const((256, 1)),
        ],
        out_specs=pl.BlockSpec(memory_space=pl.ANY),
        out_shape=jax.ShapeDtypeStruct((B, 256, S), jnp.float32),
        scratch_shapes=[
            pltpu.VMEM((nbuf, bpg, 256, S), jnp.float32),
            pltpu.SemaphoreType.DMA((nbuf,)),
        ],
    )(x, m1, b1, g1, w1, v1, m2, b2, w2, v2, w3, b3)


_S1 = np.zeros((4, 2, 2, 16), dtype=np.float32)
for _di in range(2):
    for _dj in range(2):
        for _u in range(2):
            for _v in range(2):
                _S1[_di * 2 + _dj, _u, _v, (2 * _di + _u) * 4 + (2 * _dj + _v)] = 1.0
_G1 = np.kron(np.eye(4, dtype=np.float32),
              np.full((4, 4), 0.25, dtype=np.float32))


def kernel(points, labels, boxes, masks, pe_matrix, point_embed0,
           point_embed1, point_embed2, point_embed3, not_a_point,
           conv1_w, conv1_b, ln1_w, ln1_b, conv2_w, conv2_b, ln2_w, ln2_b,
           conv3_w, conv3_b):
    B = points.shape[0]

    x = masks.reshape(B, 64, 4, 64, 4).transpose(0, 2, 4, 1, 3)
    x = x.reshape(B, 16, 4096)

    m1 = jnp.einsum("cuv,xuvr->xcr", conv1_w[:, 0], jnp.asarray(_S1))
    m1 = m1.reshape(16, 16).astype(jnp.bfloat16)
    b1 = jnp.tile(conv1_b, 4)[:, None]
    w1 = jnp.tile(ln1_w, 4)[:, None]
    v1 = jnp.tile(ln1_b, 4)[:, None]
    m2 = conv2_w.transpose(0, 2, 3, 1).reshape(16, 16).astype(jnp.bfloat16)
    b2 = conv2_b[:, None]
    w2 = ln2_w[:, None]
    v2 = ln2_b[:, None]
    w3 = conv3_w[:, :, 0, 0].astype(jnp.bfloat16)
    b3 = conv3_b[:, None]
    g1 = jnp.asarray(_G1, dtype=jnp.bfloat16)
    dense = _dense_pallas(x, m1, b1, g1, w1, v1, m2, b2, w2, v2, w3, b3)

    all_c = jnp.concatenate([points, boxes.reshape(B, 2, 2)], axis=1)
    raw = _pe_encode_pallas(all_c.reshape(B * 18, 2), pe_matrix)
    table = jnp.concatenate([not_a_point, point_embed0, point_embed1])
    badd = jnp.concatenate([point_embed2, point_embed3])
    sparse = _sc_sparse(raw.reshape(B, 18 * 256),
                        labels.astype(jnp.int32), table, badd)
    sparse = sparse.reshape(B, 18, 256)
    return sparse, dense.reshape(B, 256, 64, 64)
